# baseline (speedup 1.0000x reference)
import functools

import jax
import jax.numpy as jnp
from jax import lax
from jax.experimental import pallas as pl
from jax.experimental.pallas import tpu as pltpu
from jax.experimental.pallas import tpu_sc as plsc

VOCAB = 100000
EMBED = 128
HS1 = 1024
NCLASS = 5
B = 16384
L = 20

NC = 2
NS = 16
NW = NC * NS

BAGS_PER_W = B // NW
CHUNK_BAGS = 16
CHUNK_ROWS = CHUNK_BAGS * L
N_CHUNKS = BAGS_PER_W // CHUNK_BAGS
STREAM_IDX = 64
N_STREAMS = CHUNK_ROWS // STREAM_IDX
W_IDX_ROWS = BAGS_PER_W * L // STREAM_IDX


def _sc_pool(text3d, emb_table):
    mesh = plsc.VectorSubcoreMesh(core_axis_name="c", subcore_axis_name="s")

    @functools.partial(
        pl.kernel,
        out_type=jax.ShapeDtypeStruct((B, EMBED), jnp.float32),
        mesh=mesh,
        scratch_types=[
            pltpu.VMEM((W_IDX_ROWS, STREAM_IDX), jnp.int32),
            pltpu.VMEM((CHUNK_ROWS, EMBED), jnp.float32),
            pltpu.VMEM((CHUNK_ROWS, EMBED), jnp.float32),
            pltpu.VMEM((CHUNK_BAGS, EMBED), jnp.float32),
            pltpu.SemaphoreType.DMA,
            pltpu.SemaphoreType.DMA,
        ],
    )
    def pool_kernel(text_hbm, table_hbm, out_hbm, idx_v, rows_a, rows_b,
                    acc_v, sem_a, sem_b):
        wid = lax.axis_index("c") * NS + lax.axis_index("s")
        pltpu.sync_copy(text_hbm.at[wid], idx_v)

        def issue(ci, buf, sem):
            for j in range(N_STREAMS):
                pltpu.async_copy(
                    table_hbm.at[idx_v.at[ci * N_STREAMS + j]],
                    buf.at[pl.ds(j * STREAM_IDX, STREAM_IDX)],
                    sem,
                )

        def drain(buf, sem):
            for j in range(N_STREAMS):
                pltpu.make_async_copy(
                    table_hbm.at[pl.ds(0, STREAM_IDX)],
                    buf.at[pl.ds(j * STREAM_IDX, STREAM_IDX)],
                    sem,
                ).wait()

        def reduce_store(ci, buf):
            base_bag = wid * BAGS_PER_W + ci * CHUNK_BAGS

            def bag_fn(bi, carry2):
                r0 = bi * L
                for c8 in range(EMBED // 16):
                    sl = pl.ds(c8 * 16, 16)
                    s = buf[r0, sl]
                    for l in range(1, L):
                        s = s + buf[r0 + l, sl]
                    acc_v[bi, sl] = s
                return carry2

            lax.fori_loop(0, CHUNK_BAGS, bag_fn, 0, unroll=False)
            pltpu.sync_copy(acc_v, out_hbm.at[pl.ds(base_bag, CHUNK_BAGS)])

        issue(0, rows_a, sem_a)

        def pair_fn(pi, carry):
            c0 = pi * 2
            issue(c0 + 1, rows_b, sem_b)
            drain(rows_a, sem_a)
            reduce_store(c0, rows_a)

            @pl.when(pi + 1 < N_CHUNKS // 2)
            def _issue_next():
                issue(c0 + 2, rows_a, sem_a)

            drain(rows_b, sem_b)
            reduce_store(c0 + 1, rows_b)
            return carry

        lax.fori_loop(0, N_CHUNKS // 2, pair_fn, 0, unroll=False)

    return pool_kernel(text3d, emb_table)


def _mlp_body(x_ref, w1_ref, b1_ref, w2_ref, b2_ref, o_ref):
    x = x_ref[...].astype(jnp.bfloat16)
    h = jnp.dot(x, w1_ref[...], preferred_element_type=jnp.float32)
    h = jnp.maximum(h * (1.0 / L) + b1_ref[...], 0.0).astype(jnp.bfloat16)
    logits = (
        jnp.dot(h, w2_ref[...], preferred_element_type=jnp.float32)
        + b2_ref[...]
    )
    m = jnp.max(logits, axis=1, keepdims=True)
    e = jnp.exp(logits - m)
    o_ref[...] = (e / jnp.sum(e, axis=1, keepdims=True))[:, :8]


def _tc_mlp(pooled, w1t, b1r, w2tp, b2p):
    blk = 1024
    grid = (B // blk,)
    return pl.pallas_call(
        _mlp_body,
        grid=grid,
        in_specs=[
            pl.BlockSpec((blk, EMBED), lambda i: (i, 0)),
            pl.BlockSpec((EMBED, HS1), lambda i: (0, 0)),
            pl.BlockSpec((1, HS1), lambda i: (0, 0)),
            pl.BlockSpec((HS1, 128), lambda i: (0, 0)),
            pl.BlockSpec((1, 128), lambda i: (0, 0)),
        ],
        out_specs=pl.BlockSpec((blk, 8), lambda i: (i, 0)),
        out_shape=jax.ShapeDtypeStruct((B, 8), jnp.float32),
    )(pooled, w1t, b1r, w2tp, b2p)


def kernel(text, offsets, emb_table, W1, b1, W2, b2):
    del offsets
    text3d = text.reshape(NW, W_IDX_ROWS, STREAM_IDX)
    pooled = _sc_pool(text3d, emb_table)

    w1t = W1.T.astype(jnp.bfloat16)
    b1r = b1.reshape(1, HS1)
    w2tp = jnp.pad(W2, ((0, 128 - NCLASS), (0, 0))).T.astype(jnp.bfloat16)
    b2p = jnp.concatenate(
        [b2, jnp.full((128 - NCLASS,), -1e30, jnp.float32)]
    ).reshape(1, 128)

    out = _tc_mlp(pooled, w1t, b1r, w2tp, b2p)
    return out[:, :NCLASS]


# speedup vs baseline: 1.0161x; 1.0161x over previous
import functools

import jax
import jax.numpy as jnp
from jax import lax
from jax.experimental import pallas as pl
from jax.experimental.pallas import tpu as pltpu
from jax.experimental.pallas import tpu_sc as plsc

VOCAB = 100000
EMBED = 128
HS1 = 1024
NCLASS = 5
B = 16384
L = 20

NC = 2
NS = 16
NW = NC * NS

NHALF = 2
BH = B // NHALF
BAGS_PER_W = BH // NW
CHUNK_BAGS = 16
CHUNK_ROWS = CHUNK_BAGS * L
N_CHUNKS = BAGS_PER_W // CHUNK_BAGS
STREAM_IDX = 64
N_STREAMS = CHUNK_ROWS // STREAM_IDX
W_IDX_ROWS = BAGS_PER_W * L // STREAM_IDX


def _sc_pool_half(text4d, emb_table, half):
    mesh = plsc.VectorSubcoreMesh(core_axis_name="c", subcore_axis_name="s")

    @functools.partial(
        pl.kernel,
        out_type=jax.ShapeDtypeStruct((BH, EMBED), jnp.float32),
        mesh=mesh,
        scratch_types=[
            pltpu.VMEM((W_IDX_ROWS, STREAM_IDX), jnp.int32),
            pltpu.VMEM((CHUNK_ROWS, EMBED), jnp.float32),
            pltpu.VMEM((CHUNK_ROWS, EMBED), jnp.float32),
            pltpu.VMEM((CHUNK_BAGS, EMBED), jnp.float32),
            pltpu.SemaphoreType.DMA,
            pltpu.SemaphoreType.DMA,
        ],
    )
    def pool_kernel(text_hbm, table_hbm, out_hbm, idx_v, rows_a, rows_b,
                    acc_v, sem_a, sem_b):
        wid = lax.axis_index("c") * NS + lax.axis_index("s")
        pltpu.sync_copy(text_hbm.at[wid, half], idx_v)

        def issue(ci, buf, sem):
            for j in range(N_STREAMS):
                pltpu.async_copy(
                    table_hbm.at[idx_v.at[ci * N_STREAMS + j]],
                    buf.at[pl.ds(j * STREAM_IDX, STREAM_IDX)],
                    sem,
                )

        def drain(buf, sem):
            for j in range(N_STREAMS):
                pltpu.make_async_copy(
                    table_hbm.at[pl.ds(0, STREAM_IDX)],
                    buf.at[pl.ds(j * STREAM_IDX, STREAM_IDX)],
                    sem,
                ).wait()

        def reduce_store(ci, buf):
            base_bag = wid * BAGS_PER_W + ci * CHUNK_BAGS

            def bag_fn(bi, carry2):
                r0 = bi * L
                for c8 in range(EMBED // 16):
                    sl = pl.ds(c8 * 16, 16)
                    s = buf[r0, sl]
                    for l in range(1, L):
                        s = s + buf[r0 + l, sl]
                    acc_v[bi, sl] = s
                return carry2

            lax.fori_loop(0, CHUNK_BAGS, bag_fn, 0, unroll=False)
            pltpu.sync_copy(acc_v, out_hbm.at[pl.ds(base_bag, CHUNK_BAGS)])

        issue(0, rows_a, sem_a)

        def pair_fn(pi, carry):
            c0 = pi * 2
            issue(c0 + 1, rows_b, sem_b)
            drain(rows_a, sem_a)
            reduce_store(c0, rows_a)

            @pl.when(pi + 1 < N_CHUNKS // 2)
            def _issue_next():
                issue(c0 + 2, rows_a, sem_a)

            drain(rows_b, sem_b)
            reduce_store(c0 + 1, rows_b)
            return carry

        lax.fori_loop(0, N_CHUNKS // 2, pair_fn, 0, unroll=False)

    return pool_kernel(text4d, emb_table)


def _mlp_body(x_ref, w1_ref, b1_ref, w2_ref, b2_ref, o_ref):
    x = x_ref[...].astype(jnp.bfloat16)
    h = jnp.dot(x, w1_ref[...], preferred_element_type=jnp.float32)
    h = jnp.maximum(h * (1.0 / L) + b1_ref[...], 0.0).astype(jnp.bfloat16)
    logits = (
        jnp.dot(h, w2_ref[...], preferred_element_type=jnp.float32)
        + b2_ref[...]
    )
    m = jnp.max(logits, axis=1, keepdims=True)
    e = jnp.exp(logits - m)
    o_ref[...] = (e / jnp.sum(e, axis=1, keepdims=True))[:, :8]


def _tc_mlp(pooled, w1t, b1r, w2tp, b2p):
    blk = 1024
    nrows = pooled.shape[0]
    grid = (nrows // blk,)
    return pl.pallas_call(
        _mlp_body,
        grid=grid,
        in_specs=[
            pl.BlockSpec((blk, EMBED), lambda i: (i, 0)),
            pl.BlockSpec((EMBED, HS1), lambda i: (0, 0)),
            pl.BlockSpec((1, HS1), lambda i: (0, 0)),
            pl.BlockSpec((HS1, 128), lambda i: (0, 0)),
            pl.BlockSpec((1, 128), lambda i: (0, 0)),
        ],
        out_specs=pl.BlockSpec((blk, 8), lambda i: (i, 0)),
        out_shape=jax.ShapeDtypeStruct((nrows, 8), jnp.float32),
    )(pooled, w1t, b1r, w2tp, b2p)


def kernel(text, offsets, emb_table, W1, b1, W2, b2):
    del offsets
    text4d = text.reshape(NW, NHALF, W_IDX_ROWS, STREAM_IDX)

    w1t = W1.T.astype(jnp.bfloat16)
    b1r = b1.reshape(1, HS1)
    w2tp = jnp.pad(W2, ((0, 128 - NCLASS), (0, 0))).T.astype(jnp.bfloat16)
    b2p = jnp.concatenate(
        [b2, jnp.full((128 - NCLASS,), -1e30, jnp.float32)]
    ).reshape(1, 128)

    ys = []
    pooled = [_sc_pool_half(text4d, emb_table, k) for k in range(NHALF)]
    for k in range(NHALF):
        ys.append(_tc_mlp(pooled[k], w1t, b1r, w2tp, b2p))

    y = jnp.stack([yk.reshape(NW, BAGS_PER_W, 8) for yk in ys], axis=1)
    return y.reshape(B, 8)[:, :NCLASS]


# speedup vs baseline: 1.0175x; 1.0014x over previous
import functools

import jax
import jax.numpy as jnp
from jax import lax
from jax.experimental import pallas as pl
from jax.experimental.pallas import tpu as pltpu
from jax.experimental.pallas import tpu_sc as plsc

VOCAB = 100000
EMBED = 128
HS1 = 1024
NCLASS = 5
B = 16384
L = 20

NC = 2
NS = 16
NW = NC * NS

NHALF = 2
BH = B // NHALF
BAGS_PER_W = BH // NW
CHUNK_BAGS = 16
CHUNK_ROWS = CHUNK_BAGS * L
N_CHUNKS = BAGS_PER_W // CHUNK_BAGS
STREAM_IDX = 64
N_STREAMS = CHUNK_ROWS // STREAM_IDX
W_IDX_ROWS = BAGS_PER_W * L // 128


def _sc_pool_half(text4d, emb_table, half):
    mesh = plsc.VectorSubcoreMesh(core_axis_name="c", subcore_axis_name="s")

    @functools.partial(
        pl.kernel,
        out_type=jax.ShapeDtypeStruct((BH, EMBED), jnp.float32),
        mesh=mesh,
        scratch_types=[
            pltpu.VMEM((W_IDX_ROWS, 128), jnp.int32),
            pltpu.VMEM((CHUNK_ROWS, EMBED), jnp.float32),
            pltpu.VMEM((CHUNK_ROWS, EMBED), jnp.float32),
            pltpu.VMEM((CHUNK_BAGS, EMBED), jnp.float32),
            pltpu.SemaphoreType.DMA,
            pltpu.SemaphoreType.DMA,
        ],
    )
    def pool_kernel(text_hbm, table_hbm, out_hbm, idx_v, rows_a, rows_b,
                    acc_v, sem_a, sem_b):
        wid = lax.axis_index("c") * NS + lax.axis_index("s")
        pltpu.sync_copy(text_hbm.at[wid, half], idx_v)

        def issue(ci, buf, sem):
            for j in range(N_STREAMS):
                flat = ci * CHUNK_ROWS + j * STREAM_IDX
                row, col = flat // 128, flat % 128
                pltpu.async_copy(
                    table_hbm.at[idx_v.at[row, pl.ds(col, STREAM_IDX)]],
                    buf.at[pl.ds(j * STREAM_IDX, STREAM_IDX)],
                    sem,
                )

        def drain(buf, sem):
            for j in range(N_STREAMS):
                pltpu.make_async_copy(
                    table_hbm.at[pl.ds(0, STREAM_IDX)],
                    buf.at[pl.ds(j * STREAM_IDX, STREAM_IDX)],
                    sem,
                ).wait()

        def reduce_store(ci, buf):
            base_bag = wid * BAGS_PER_W + ci * CHUNK_BAGS

            def bag_fn(bi, carry2):
                r0 = bi * L
                for c8 in range(EMBED // 16):
                    sl = pl.ds(c8 * 16, 16)
                    s = buf[r0, sl]
                    for l in range(1, L):
                        s = s + buf[r0 + l, sl]
                    acc_v[bi, sl] = s
                return carry2

            lax.fori_loop(0, CHUNK_BAGS, bag_fn, 0, unroll=False)
            pltpu.sync_copy(acc_v, out_hbm.at[pl.ds(base_bag, CHUNK_BAGS)])

        issue(0, rows_a, sem_a)

        def pair_fn(pi, carry):
            c0 = pi * 2
            issue(c0 + 1, rows_b, sem_b)
            drain(rows_a, sem_a)
            reduce_store(c0, rows_a)

            @pl.when(pi + 1 < N_CHUNKS // 2)
            def _issue_next():
                issue(c0 + 2, rows_a, sem_a)

            drain(rows_b, sem_b)
            reduce_store(c0 + 1, rows_b)
            return carry

        lax.fori_loop(0, N_CHUNKS // 2, pair_fn, 0, unroll=False)

    return pool_kernel(text4d, emb_table)


def _mlp_body(x_ref, w1_ref, b1_ref, w2_ref, b2_ref, o_ref):
    x = x_ref[...].astype(jnp.bfloat16)
    h = jnp.dot(x, w1_ref[...], preferred_element_type=jnp.float32)
    h = jnp.maximum(h * (1.0 / L) + b1_ref[...], 0.0).astype(jnp.bfloat16)
    logits = (
        jnp.dot(h, w2_ref[...], preferred_element_type=jnp.float32)
        + b2_ref[...]
    )
    m = jnp.max(logits, axis=1, keepdims=True)
    e = jnp.exp(logits - m)
    o_ref[...] = (e / jnp.sum(e, axis=1, keepdims=True))[:, :8]


def _tc_mlp(pooled, w1t, b1r, w2tp, b2p):
    blk = 1024
    nrows = pooled.shape[0]
    grid = (nrows // blk,)
    return pl.pallas_call(
        _mlp_body,
        grid=grid,
        in_specs=[
            pl.BlockSpec((blk, EMBED), lambda i: (i, 0)),
            pl.BlockSpec((EMBED, HS1), lambda i: (0, 0)),
            pl.BlockSpec((1, HS1), lambda i: (0, 0)),
            pl.BlockSpec((HS1, 128), lambda i: (0, 0)),
            pl.BlockSpec((1, 128), lambda i: (0, 0)),
        ],
        out_specs=pl.BlockSpec((blk, 8), lambda i: (i, 0)),
        out_shape=jax.ShapeDtypeStruct((nrows, 8), jnp.float32),
    )(pooled, w1t, b1r, w2tp, b2p)


def kernel(text, offsets, emb_table, W1, b1, W2, b2):
    del offsets
    text4d = text.reshape(NW, NHALF, W_IDX_ROWS, 128)

    w1t = W1.T.astype(jnp.bfloat16)
    b1r = b1.reshape(1, HS1)
    w2tp = jnp.pad(W2, ((0, 128 - NCLASS), (0, 0))).T.astype(jnp.bfloat16)
    b2p = jnp.concatenate(
        [b2, jnp.full((128 - NCLASS,), -1e30, jnp.float32)]
    ).reshape(1, 128)

    ys = []
    pooled = [_sc_pool_half(text4d, emb_table, k) for k in range(NHALF)]
    for k in range(NHALF):
        ys.append(_tc_mlp(pooled[k], w1t, b1r, w2tp, b2p))

    y = jnp.stack([yk.reshape(NW, BAGS_PER_W, 8) for yk in ys], axis=1)
    return y.reshape(B, 8)[:, :NCLASS]
